# baseline (speedup 1.0000x reference)
import functools

import jax
import jax.numpy as jnp
from jax import lax
from jax.experimental import pallas as pl
from jax.experimental.pallas import tpu as pltpu, tpu_sc as plsc

_N_CLASSES = 1000
_B, _T = 4096, 26
_NC, _NS = 2, 16
_W_WORDS = _T * _N_CLASSES * _B
_CORE_WORDS = _W_WORDS // _NC
_SUB_WORDS = _CORE_WORDS // _NS
_CHUNK = 83_200
_N_CHUNK = _SUB_WORDS // _CHUNK
_T_PER_CORE = _T // _NC
_B_PER_SUB = _B // _NS
_N_ONES = _T_PER_CORE * _B_PER_SUB

_mesh = plsc.VectorSubcoreMesh(core_axis_name="c", subcore_axis_name="s")


@functools.partial(
    pl.kernel,
    mesh=_mesh,
    out_type=jax.ShapeDtypeStruct((_W_WORDS,), jnp.float32),
    scratch_types=[
        pltpu.VMEM((_CHUNK,), jnp.float32),
        pltpu.VMEM((_B_PER_SUB,), jnp.int32),
        pltpu.VMEM((_N_ONES,), jnp.int32),
        pltpu.VMEM((_N_ONES,), jnp.float32),
        pltpu.SemaphoreType.DMA,
    ],
)
def _sc_onehot(idxt_hbm, out_hbm, zeros_v, row_v, offs_v, ones_v, sem):
    c = lax.axis_index("c")
    s = lax.axis_index("s")

    @pl.loop(0, _CHUNK // 16)
    def _zinit(i):
        zeros_v[pl.ds(i * 16, 16)] = jnp.zeros((16,), jnp.float32)

    @pl.loop(0, _N_ONES // 16)
    def _oinit(i):
        ones_v[pl.ds(i * 16, 16)] = jnp.full((16,), 1.0, jnp.float32)

    base = c * _CORE_WORDS + s * _SUB_WORDS

    @pl.loop(0, _N_CHUNK)
    def _zfill(i):
        pltpu.sync_copy(zeros_v, out_hbm.at[pl.ds(base + i * _CHUNK, _CHUNK)])

    t0 = c * _T_PER_CORE
    b0 = s * _B_PER_SUB
    lane = lax.iota(jnp.int32, 16)

    @pl.loop(0, _T_PER_CORE)
    def _per_t(t):
        pltpu.sync_copy(
            idxt_hbm.at[pl.ds((t0 + t) * _B + b0, _B_PER_SUB)], row_v
        )

        @pl.loop(0, _B_PER_SUB // 16)
        def _per_vec(k):
            iv = row_v[pl.ds(k * 16, 16)]
            off = iv * _B + ((t0 + t) * _N_CLASSES * _B + b0 + k * 16) + lane
            offs_v[pl.ds(t * _B_PER_SUB + k * 16, 16)] = off

    plsc.subcore_barrier()
    pltpu.async_copy(ones_v, out_hbm.at[offs_v], sem).wait()


def kernel(input):
    idx_flat = input.astype(jnp.int32).T.reshape(_T * _B)
    out = _sc_onehot(idx_flat)
    return out.reshape(_T, _N_CLASSES, _B).transpose(2, 0, 1)


# speedup vs baseline: 5.4647x; 5.4647x over previous
import jax
import jax.numpy as jnp
from jax.experimental import pallas as pl

_N_CLASSES = 1000
_B, _T = 4096, 26
_B_BLK = 2048
_T_BLK = 1


def _onehot_block(idx_ref, out_ref):
    idx = idx_ref[...]
    iota = jax.lax.broadcasted_iota(
        jnp.int32, (_T_BLK, _N_CLASSES, _B_BLK), 1
    )
    out_ref[...] = (iota == idx).astype(jnp.float32)


def kernel(input):
    idx_t = input.astype(jnp.int32).T.reshape(_T, 1, _B)
    out = pl.pallas_call(
        _onehot_block,
        grid=(_T // _T_BLK, _B // _B_BLK),
        in_specs=[pl.BlockSpec((_T_BLK, 1, _B_BLK), lambda t, j: (t, 0, j))],
        out_specs=pl.BlockSpec(
            (_T_BLK, _N_CLASSES, _B_BLK), lambda t, j: (t, 0, j)
        ),
        out_shape=jax.ShapeDtypeStruct((_T, _N_CLASSES, _B), jnp.float32),
    )(idx_t)
    return out.transpose(2, 0, 1)
